# Initial kernel scaffold; baseline (speedup 1.0000x reference)
import jax
import jax.numpy as jnp
from jax.experimental import pallas as pl


def kernel(input, codes, codebooks):
    raise NotImplementedError("write your pallas kernel here")


# speedup vs baseline: 28.9854x; 28.9854x over previous
import functools

import jax
import jax.numpy as jnp
from jax import lax
from jax.experimental import pallas as pl
from jax.experimental.pallas import tpu as pltpu
from jax.experimental.pallas import tpu_sc as plsc

N_EMB = 1000000
DIM = 64
NCB = 8
CB_SIZE = 256
CB_DIM = 8
BATCH = 4096
HIST = 200
N = BATCH * HIST

NW = 32
PER_W = N // NW
SUB = 128
CHUNK = 1024
NSUB = CHUNK // SUB
NCHUNK = PER_W // CHUNK


def _build_sc_kernel():
    mesh = plsc.VectorSubcoreMesh(core_axis_name="c", subcore_axis_name="s")

    @functools.partial(
        pl.kernel,
        out_type=jax.ShapeDtypeStruct((N * DIM,), jnp.float32),
        mesh=mesh,
        scratch_types=[
            pltpu.VMEM((NCB * CB_SIZE * CB_DIM,), jnp.float32),
            pltpu.VMEM((NSUB, SUB), jnp.int32),
            pltpu.VMEM((CHUNK, NCB), jnp.int32),
            pltpu.VMEM((CHUNK * DIM,), jnp.float32),
            pltpu.SemaphoreType.DMA,
        ],
        compiler_params=pltpu.CompilerParams(
            needs_layout_passes=False, use_tc_tiling_on_sc=False),
    )
    def rpq_sc(idx_hbm, codes_t_hbm, cb_hbm, out_hbm,
               cb_v, idx_v, codes_v, out_v, sem):
        wid = lax.axis_index("c") * 16 + lax.axis_index("s")
        row_base = wid * PER_W
        sub_base = row_base // SUB

        pltpu.sync_copy(cb_hbm, cb_v)

        iota16 = lax.iota(jnp.int32, 16)

        @pl.loop(0, NCHUNK)
        def chunk_loop(g):
            row0 = row_base + g * CHUNK
            sub_off = pl.multiple_of(sub_base + g * NSUB, 8)
            pltpu.sync_copy(idx_hbm.at[pl.ds(sub_off, NSUB)], idx_v)
            copies = [
                pltpu.async_copy(codes_t_hbm.at[idx_v.at[j]],
                                 codes_v.at[pl.ds(j * SUB, SUB)], sem)
                for j in range(NSUB)
            ]
            for c in copies:
                c.wait()

            @pl.loop(0, CHUNK // 16)
            def row_loop(t):
                rvec = t * 16 + iota16
                ovec = rvec * DIM
                for h in range(NCB):
                    hvec = jnp.full((16,), h, jnp.int32)
                    code = plsc.load_gather(codes_v, [rvec, hvec])
                    cbase = code * CB_DIM + (h * CB_SIZE * CB_DIM)
                    obase = ovec + (h * CB_DIM)
                    for d in range(CB_DIM):
                        val = plsc.load_gather(cb_v, [cbase + d])
                        plsc.store_scatter(out_v, [obase + d], val)

            out_off = pl.multiple_of(row0 * DIM, 8)
            pltpu.sync_copy(out_v, out_hbm.at[pl.ds(out_off, CHUNK * DIM)])

    return rpq_sc


_RPQ_SC = _build_sc_kernel()


@jax.jit
def kernel(input, codes, codebooks):
    idx = input.reshape(N // SUB, SUB)
    codes_t = jnp.swapaxes(codes, 0, 1)
    cb_flat = codebooks.reshape(-1)
    out = _RPQ_SC(idx, codes_t, cb_flat)
    return out.reshape(input.shape + (DIM,))


# speedup vs baseline: 30.2569x; 1.0439x over previous
import functools

import jax
import jax.numpy as jnp
from jax import lax
from jax.experimental import pallas as pl
from jax.experimental.pallas import tpu as pltpu
from jax.experimental.pallas import tpu_sc as plsc

N_EMB = 1000000
DIM = 64
NCB = 8
CB_SIZE = 256
CB_DIM = 8
BATCH = 4096
HIST = 200
N = BATCH * HIST

NW = 32
PER_W = N // NW
SUB = 128
CHUNK = 1024
HALF = CHUNK // 2
NSUB = CHUNK // SUB
NCHUNK = PER_W // CHUNK


def _build_sc_kernel():
    mesh = plsc.VectorSubcoreMesh(core_axis_name="c", subcore_axis_name="s")

    @functools.partial(
        pl.kernel,
        out_type=jax.ShapeDtypeStruct((N * DIM,), jnp.float32),
        mesh=mesh,
        scratch_types=[
            pltpu.VMEM((NCB * CB_SIZE * CB_DIM,), jnp.float32),
            pltpu.VMEM((2, NSUB, SUB), jnp.int32),
            pltpu.VMEM((2, CHUNK, NCB), jnp.int32),
            pltpu.VMEM((2, HALF * DIM), jnp.float32),
            pltpu.SemaphoreType.DMA,
            pltpu.SemaphoreType.DMA,
            pltpu.SemaphoreType.DMA,
        ],
        compiler_params=pltpu.CompilerParams(
            needs_layout_passes=False, use_tc_tiling_on_sc=False),
    )
    def rpq_sc(idx_hbm, codes_t_hbm, cb_hbm, out_hbm,
               cb_v, idx_v, codes_v, out_v, sem_g, sem_o0, sem_o1):
        wid = lax.axis_index("c") * 16 + lax.axis_index("s")
        row_base = wid * PER_W
        sub_base = row_base // SUB
        sem_o = (sem_o0, sem_o1)

        pltpu.sync_copy(cb_hbm, cb_v)

        iota16 = lax.iota(jnp.int32, 16)

        def fetch(g, slot):
            sub_off = pl.multiple_of(sub_base + g * NSUB, 8)
            pltpu.sync_copy(idx_hbm.at[pl.ds(sub_off, NSUB)], idx_v.at[slot])
            for j in range(NSUB):
                pltpu.async_copy(codes_t_hbm.at[idx_v.at[slot, j]],
                                 codes_v.at[slot, pl.ds(j * SUB, SUB)],
                                 sem_g)

        fetch(0, 0)

        @pl.loop(0, NCHUNK)
        def chunk_loop(g):
            p = lax.rem(g, 2)
            pltpu.make_async_copy(codes_t_hbm.at[pl.ds(0, CHUNK)],
                                  codes_v.at[p], sem_g).wait()

            @pl.when(g + 1 < NCHUNK)
            def _():
                fetch(g + 1, 1 - p)

            pvec = jnp.broadcast_to(p, (16,))
            for k in range(2):
                @pl.when(g > 0)
                def _():
                    pltpu.make_async_copy(
                        out_hbm.at[pl.ds(0, HALF * DIM)],
                        out_v.at[k], sem_o[k]).wait()

                @pl.loop(0, HALF // 16)
                def row_loop(t):
                    rloc = t * 16 + iota16
                    rvec = k * HALF + rloc
                    ovec = rloc * DIM
                    for h in range(NCB):
                        hvec = jnp.full((16,), h, jnp.int32)
                        code = plsc.load_gather(codes_v, [pvec, rvec, hvec])
                        cbase = code * CB_DIM + (h * CB_SIZE * CB_DIM)
                        obase = ovec + (h * CB_DIM)
                        for d in range(CB_DIM):
                            val = plsc.load_gather(cb_v, [cbase + d])
                            plsc.store_scatter(out_v.at[k], [obase + d], val)

                out_off = pl.multiple_of(
                    (row_base + g * CHUNK + k * HALF) * DIM, 8)
                pltpu.async_copy(out_v.at[k],
                                 out_hbm.at[pl.ds(out_off, HALF * DIM)],
                                 sem_o[k])

        for k in range(2):
            pltpu.make_async_copy(out_hbm.at[pl.ds(0, HALF * DIM)],
                                  out_v.at[k], sem_o[k]).wait()

    return rpq_sc


_RPQ_SC = _build_sc_kernel()


@jax.jit
def kernel(input, codes, codebooks):
    idx = input.reshape(N // SUB, SUB)
    codes_t = jnp.swapaxes(codes, 0, 1)
    cb_flat = codebooks.reshape(-1)
    out = _RPQ_SC(idx, codes_t, cb_flat)
    return out.reshape(input.shape + (DIM,))


# speedup vs baseline: 57.9990x; 1.9169x over previous
import functools

import jax
import jax.numpy as jnp
from jax import lax
from jax.experimental import pallas as pl
from jax.experimental.pallas import tpu as pltpu
from jax.experimental.pallas import tpu_sc as plsc

N_EMB = 1000000
DIM = 64
NCB = 8
CB_SIZE = 256
CB_DIM = 8
BATCH = 4096
HIST = 200
N = BATCH * HIST

NW = 32
PER_W = N // NW
SUB = 128
CHUNK = 1024
HALF = CHUNK // 2
NSUB = CHUNK // SUB
NCHUNK = PER_W // CHUNK
G2 = HALF * NCB // SUB

PREP_BLK = 4096


def _prep_body(codes_ref, out_ref):
    x = codes_ref[...]
    offs = lax.broadcasted_iota(jnp.int32, (NCB, PREP_BLK), 0) * CB_SIZE
    out_ref[...] = (x + offs).T


_PREP = pl.pallas_call(
    _prep_body,
    grid=((N_EMB + PREP_BLK - 1) // PREP_BLK,),
    in_specs=[pl.BlockSpec((NCB, PREP_BLK), lambda i: (0, i))],
    out_specs=pl.BlockSpec((PREP_BLK, NCB), lambda i: (i, 0)),
    out_shape=jax.ShapeDtypeStruct((N_EMB, NCB), jnp.int32),
)


def _build_sc_kernel():
    mesh = plsc.VectorSubcoreMesh(core_axis_name="c", subcore_axis_name="s")

    @functools.partial(
        pl.kernel,
        out_type=jax.ShapeDtypeStruct((N * NCB, CB_DIM), jnp.float32),
        mesh=mesh,
        scratch_types=[
            pltpu.VMEM((2, NSUB, SUB), jnp.int32),
            pltpu.VMEM((2, CHUNK, NCB), jnp.int32),
            pltpu.VMEM((CHUNK * NCB,), jnp.int32),
            pltpu.VMEM((2, HALF * NCB, CB_DIM), jnp.float32),
            pltpu.VMEM_SHARED((NCB * CB_SIZE, CB_DIM), jnp.float32),
            pltpu.SemaphoreType.DMA,
            pltpu.SemaphoreType.DMA,
            pltpu.SemaphoreType.DMA,
            pltpu.SemaphoreType.DMA,
            pltpu.SemaphoreType.DMA,
        ],
        compiler_params=pltpu.CompilerParams(
            needs_layout_passes=False, use_tc_tiling_on_sc=False),
    )
    def rpq_sc(idx_hbm, codes_plus_hbm, cb_hbm, out_hbm,
               idx_v, il_v, fl_v, out_v, cb_sh,
               sem_g, sem_c0, sem_c1, sem_o0, sem_o1):
        wid = lax.axis_index("c") * 16 + lax.axis_index("s")
        row_base = wid * PER_W
        sub_base = row_base // SUB
        sem_c = (sem_c0, sem_c1)
        sem_o = (sem_o0, sem_o1)

        @pl.when(lax.axis_index("s") == 0)
        def _():
            pltpu.sync_copy(cb_hbm, cb_sh)

        plsc.subcore_barrier()

        iota16 = lax.iota(jnp.int32, 16)

        def fetch(g, slot):
            sub_off = pl.multiple_of(sub_base + g * NSUB, 8)
            pltpu.sync_copy(idx_hbm.at[pl.ds(sub_off, NSUB)], idx_v.at[slot])
            for j in range(NSUB):
                pltpu.async_copy(codes_plus_hbm.at[idx_v.at[slot, j]],
                                 il_v.at[slot, pl.ds(j * SUB, SUB)],
                                 sem_g)

        fetch(0, 0)

        @pl.loop(0, NCHUNK)
        def chunk_loop(g):
            p = lax.rem(g, 2)
            pltpu.make_async_copy(codes_plus_hbm.at[pl.ds(0, CHUNK)],
                                  il_v.at[p], sem_g).wait()

            @pl.when(g + 1 < NCHUNK)
            def _():
                fetch(g + 1, 1 - p)

            pvec = jnp.broadcast_to(p, (16,))
            for k in range(2):
                @pl.when(g > 0)
                def _():
                    pltpu.make_async_copy(
                        out_hbm.at[pl.ds(0, HALF * NCB)],
                        out_v.at[k], sem_o[k]).wait()

                @pl.loop(0, HALF // 16)
                def row_loop(t):
                    rvec = k * HALF + t * 16 + iota16
                    rvec8 = rvec * NCB
                    vals = []
                    for h in range(NCB):
                        hvec = jnp.full((16,), h, jnp.int32)
                        vals.append(
                            plsc.load_gather(il_v, [pvec, rvec, hvec]))
                    for h in range(NCB):
                        plsc.store_scatter(fl_v, [rvec8 + h], vals[h])

                for j in range(G2):
                    pltpu.async_copy(
                        cb_sh.at[fl_v.at[pl.ds(k * HALF * NCB + j * SUB,
                                               SUB)]],
                        out_v.at[k, pl.ds(j * SUB, SUB)],
                        sem_c[k])

            for k in range(2):
                pltpu.make_async_copy(out_hbm.at[pl.ds(0, HALF * NCB)],
                                      out_v.at[k], sem_c[k]).wait()
                out_off = pl.multiple_of(
                    (row_base + g * CHUNK + k * HALF) * NCB, 8)
                pltpu.async_copy(out_v.at[k],
                                 out_hbm.at[pl.ds(out_off, HALF * NCB)],
                                 sem_o[k])

        for k in range(2):
            pltpu.make_async_copy(out_hbm.at[pl.ds(0, HALF * NCB)],
                                  out_v.at[k], sem_o[k]).wait()

    return rpq_sc


_RPQ_SC = _build_sc_kernel()


@jax.jit
def kernel(input, codes, codebooks):
    idx = input.reshape(N // SUB, SUB)
    codes_plus = _PREP(codes)
    cb2 = codebooks.reshape(NCB * CB_SIZE, CB_DIM)
    out = _RPQ_SC(idx, codes_plus, cb2)
    return out.reshape(input.shape + (DIM,))
